# baseline (speedup 1.0000x reference)
import functools

import jax
import jax.numpy as jnp
from jax import lax
from jax.experimental import pallas as pl
from jax.experimental.pallas import tpu as pltpu
from jax.experimental.pallas import tpu_sc as plsc


@functools.lru_cache(maxsize=None)
def _make_gather(V, D, B, C):
    info = plsc.get_sparse_core_info()
    NC, NS = info.num_cores, info.num_subcores
    NW = NC * NS
    assert B % (NW * C * 2) == 0
    b_per_w = B // NW
    n_chunks = b_per_w // C

    mesh = plsc.VectorSubcoreMesh(core_axis_name="c", subcore_axis_name="s")

    @functools.partial(
        pl.kernel,
        mesh=mesh,
        out_type=jax.ShapeDtypeStruct((B, D), jnp.float32),
        scratch_types=[
            pltpu.VMEM((C,), jnp.int32),
            pltpu.VMEM((C,), jnp.int32),
            pltpu.VMEM((C, D), jnp.float32),
            pltpu.VMEM((C, D), jnp.float32),
            pltpu.SemaphoreType.DMA((2,)),
            pltpu.SemaphoreType.DMA((2,)),
        ],
        compiler_params=pltpu.CompilerParams(use_tc_tiling_on_sc=False),
    )
    def k(tab_hbm, idx_hbm, out_hbm, i0, i1, r0, r1, gsem, ssem):
        idx_v = [i0, i1]
        rows_v = [r0, r1]
        wid = lax.axis_index("s") * NC + lax.axis_index("c")
        base = wid * b_per_w

        def start_gather(i, p):
            pltpu.sync_copy(idx_hbm.at[pl.ds(base + i * C, C)], idx_v[p])
            pltpu.async_copy(tab_hbm.at[idx_v[p]], rows_v[p], gsem.at[p])

        def wait_gather(p):
            pltpu.make_async_copy(
                tab_hbm.at[idx_v[p]], rows_v[p], gsem.at[p]
            ).wait()

        def start_store(i, p):
            pltpu.async_copy(
                rows_v[p], out_hbm.at[pl.ds(base + i * C, C)], ssem.at[p]
            )

        def wait_store(i, p):
            pltpu.make_async_copy(
                rows_v[p], out_hbm.at[pl.ds(base + i * C, C)], ssem.at[p]
            ).wait()

        start_gather(0, 0)

        def body(j, carry):
            for p in range(2):
                i = j * 2 + p

                @pl.when(i + 1 < n_chunks)
                def _prefetch():
                    start_gather(i + 1, 1 - p)

                wait_gather(p)

                @pl.when(i >= 2)
                def _drain():
                    wait_store(i - 2, p)

                start_store(i, p)
            return carry

        lax.fori_loop(0, n_chunks // 2, body, 0)
        for i in range(n_chunks - 2, n_chunks):
            wait_store(i, i % 2)

    return k


def kernel(x, table):
    BATCH, HIST = x.shape
    V, D = table.shape
    B = BATCH * HIST
    xf = x.reshape(B).astype(jnp.int32)
    out = _make_gather(V, D, B, 200)(table, xf)
    return out.reshape(BATCH, HIST, D)
